# baseline (speedup 1.0000x reference)
import functools
import math

import jax
import jax.numpy as jnp
from jax import lax
from jax.experimental import pallas as pl
from jax.experimental.pallas import tpu as pltpu
from jax.experimental.pallas import tpu_sc as plsc

D_MODEL = 128
SCALE = math.sqrt(D_MODEL)

NUM_CORES = 2
NUM_SUBCORES = 16
NW = NUM_CORES * NUM_SUBCORES
LANES = 16

CHUNK = 128
B_TOTAL = 4096 * 200
B_PER_W = B_TOTAL // NW
CHUNKS_PER_W = B_PER_W // CHUNK
NBUF = 5
LOOKAHEAD = 2
QUADS = CHUNKS_PER_W // NBUF


@functools.partial(
    pl.kernel,
    mesh=plsc.VectorSubcoreMesh(core_axis_name="c", subcore_axis_name="s"),
    out_type=jax.ShapeDtypeStruct((B_TOTAL, D_MODEL), jnp.float32),
    scratch_types=[
        pltpu.VMEM((CHUNKS_PER_W, CHUNK), jnp.int32),
    ] + [pltpu.VMEM((CHUNK, D_MODEL), jnp.float32)] * NBUF
      + [pltpu.SemaphoreType.DMA] * (2 * NBUF),
)
def _emb_lookup(x_hbm, table_hbm, out_hbm, idx_v, b0, b1, b2, b3, b4,
                g0, g1, g2, g3, g4, s0, s1, s2, s3, s4):
    bufs = (b0, b1, b2, b3, b4)
    gsems = (g0, g1, g2, g3, g4)
    ssems = (s0, s1, s2, s3, s4)

    wid = lax.axis_index("s") * NUM_CORES + lax.axis_index("c")
    base = wid * B_PER_W

    pltpu.sync_copy(x_hbm.at[wid], idx_v)

    def gather_start(g, buf, sem):
        pltpu.async_copy(table_hbm.at[idx_v.at[g]], buf, sem)

    def gather_wait(g, buf, sem):
        pltpu.make_async_copy(table_hbm.at[idx_v.at[g]], buf, sem).wait()

    def scatter_start(g, buf, sem):
        pltpu.async_copy(buf, out_hbm.at[pl.ds(base + g * CHUNK, CHUNK)], sem)

    def scatter_wait(g, buf, sem):
        pltpu.make_async_copy(
            buf, out_hbm.at[pl.ds(base + g * CHUNK, CHUNK)], sem).wait()

    def scale(buf):
        def row_body(r, carry):
            for j in range(D_MODEL // LANES):
                sl = pl.ds(j * LANES, LANES)
                buf[r, sl] = buf[r, sl] * SCALE
            return carry
        lax.fori_loop(0, CHUNK, row_body, 0, unroll=2)

    for g in range(LOOKAHEAD):
        gather_start(g, bufs[g], gsems[g])

    def quad_body(p, carry):
        for b in range(NBUF):
            g = NBUF * p + b
            bb = (b + LOOKAHEAD) % NBUF

            @pl.when(g + LOOKAHEAD < CHUNKS_PER_W)
            def _():
                @pl.when(g >= NBUF - LOOKAHEAD)
                def _():
                    scatter_wait(g + LOOKAHEAD - NBUF, bufs[bb], ssems[bb])
                gather_start(g + LOOKAHEAD, bufs[bb], gsems[bb])

            gather_wait(g, bufs[b], gsems[b])
            scale(bufs[b])
            scatter_start(g, bufs[b], ssems[b])
        return carry

    lax.fori_loop(0, QUADS, quad_body, 0, unroll=False)

    for g in range(CHUNKS_PER_W - NBUF, CHUNKS_PER_W):
        b = g % NBUF
        scatter_wait(g, bufs[b], ssems[b])


def kernel(x, table):
    xf = x.reshape(NW, CHUNKS_PER_W, CHUNK).astype(jnp.int32)
    out = _emb_lookup(xf, table)
    return out.reshape(x.shape[0], x.shape[1], D_MODEL)


# speedup vs baseline: 1.0023x; 1.0023x over previous
import functools
import math

import jax
import jax.numpy as jnp
from jax import lax
from jax.experimental import pallas as pl
from jax.experimental.pallas import tpu as pltpu
from jax.experimental.pallas import tpu_sc as plsc

D_MODEL = 128
SCALE = math.sqrt(D_MODEL)

NUM_CORES = 2
NUM_SUBCORES = 16
NW = NUM_CORES * NUM_SUBCORES
LANES = 16

CHUNK = 128
B_TOTAL = 4096 * 200
B_PER_W = B_TOTAL // NW
CHUNKS_PER_W = B_PER_W // CHUNK
NBUF = 4
LOOKAHEAD = 2
QUADS = CHUNKS_PER_W // NBUF


@functools.partial(
    pl.kernel,
    mesh=plsc.VectorSubcoreMesh(core_axis_name="c", subcore_axis_name="s"),
    out_type=jax.ShapeDtypeStruct((B_TOTAL, D_MODEL), jnp.float32),
    scratch_types=[
        pltpu.VMEM((CHUNKS_PER_W, CHUNK), jnp.int32),
    ] + [pltpu.VMEM((CHUNK, D_MODEL), jnp.float32)] * NBUF
      + [pltpu.SemaphoreType.DMA] * (2 * NBUF),
)
def _emb_lookup(x_hbm, table_hbm, out_hbm, idx_v, b0, b1, b2, b3,
                g0, g1, g2, g3, s0, s1, s2, s3):
    bufs = (b0, b1, b2, b3)
    gsems = (g0, g1, g2, g3)
    ssems = (s0, s1, s2, s3)

    wid = lax.axis_index("s") * NUM_CORES + lax.axis_index("c")
    base = wid * B_PER_W

    pltpu.sync_copy(x_hbm.at[wid], idx_v)

    def gather_start(g, buf, sem):
        pltpu.async_copy(table_hbm.at[idx_v.at[g]], buf, sem)

    def gather_wait(g, buf, sem):
        pltpu.make_async_copy(table_hbm.at[idx_v.at[g]], buf, sem).wait()

    def scatter_start(g, buf, sem):
        pltpu.async_copy(buf, out_hbm.at[pl.ds(base + g * CHUNK, CHUNK)], sem)

    def scatter_wait(g, buf, sem):
        pltpu.make_async_copy(
            buf, out_hbm.at[pl.ds(base + g * CHUNK, CHUNK)], sem).wait()

    def scale(buf):
        def row_body(r, carry):
            for j in range(D_MODEL // LANES):
                sl = pl.ds(j * LANES, LANES)
                buf[r, sl] = buf[r, sl] * SCALE
            return carry
        lax.fori_loop(0, CHUNK, row_body, 0, unroll=2)

    for g in range(LOOKAHEAD):
        gather_start(g, bufs[g], gsems[g])

    def quad_body(p, carry):
        for b in range(NBUF):
            g = NBUF * p + b
            bb = (b + LOOKAHEAD) % NBUF

            @pl.when(g + LOOKAHEAD < CHUNKS_PER_W)
            def _():
                @pl.when(g >= NBUF - LOOKAHEAD)
                def _():
                    scatter_wait(g + LOOKAHEAD - NBUF, bufs[bb], ssems[bb])
                gather_start(g + LOOKAHEAD, bufs[bb], gsems[bb])

            gather_wait(g, bufs[b], gsems[b])
            scale(bufs[b])
            scatter_start(g, bufs[b], ssems[b])
        return carry

    lax.fori_loop(0, QUADS, quad_body, 0, unroll=False)

    for g in range(CHUNKS_PER_W - NBUF, CHUNKS_PER_W):
        b = g % NBUF
        scatter_wait(g, bufs[b], ssems[b])


def kernel(x, table):
    xf = x.reshape(NW, CHUNKS_PER_W, CHUNK).astype(jnp.int32)
    out = _emb_lookup(xf, table)
    return out.reshape(x.shape[0], x.shape[1], D_MODEL)
